# Initial kernel scaffold; baseline (speedup 1.0000x reference)
import jax
import jax.numpy as jnp
from jax.experimental import pallas as pl


def kernel(sequence, sequence_segment, token_table, seg_table, W, b, gamma, beta):
    raise NotImplementedError("write your pallas kernel here")


# speedup vs baseline: 1.8816x; 1.8816x over previous
import functools

import jax
import jax.numpy as jnp
import numpy as np
from jax import lax
from jax.experimental import pallas as pl
from jax.experimental.pallas import tpu as pltpu
from jax.experimental.pallas import tpu_sc as plsc

_VOCAB = 100000
_EMBED = 128
_DMODEL = 768
_MAXLEN = 2048
_EPS = 1e-5


def _sinusoidal_pe_np(max_len, d):
    pos = np.arange(max_len, dtype=np.float32)[:, None]
    div = np.exp(np.arange(0, d, 2, dtype=np.float32) * (-np.log(10000.0) / d))
    pe = np.zeros((max_len, d), dtype=np.float32)
    pe[:, 0::2] = np.sin(pos * div)
    pe[:, 1::2] = np.cos(pos * div)
    return pe


def _sc_gather(token_table, idx_2d, n_rows):
    info = plsc.get_sparse_core_info()
    nc, ns = info.num_cores, info.num_subcores
    nw = nc * ns
    rows_per_w = n_rows // nw
    chunks_per_w = rows_per_w // 128

    mesh = plsc.VectorSubcoreMesh(core_axis_name="c", subcore_axis_name="s")

    @functools.partial(
        pl.kernel,
        mesh=mesh,
        out_type=jax.ShapeDtypeStruct((n_rows, _EMBED), jnp.float32),
        scratch_types=[
            pltpu.VMEM((chunks_per_w, 128), jnp.int32),
            pltpu.VMEM((rows_per_w, _EMBED), jnp.float32),
            pltpu.SemaphoreType.DMA,
        ],
    )
    def gather_kernel(table_hbm, idx_hbm, out_hbm, idx_v, rows_v, sem):
        wid = lax.axis_index("s") * nc + lax.axis_index("c")
        pltpu.sync_copy(idx_hbm.at[pl.ds(wid * chunks_per_w, chunks_per_w)], idx_v)
        copies = []
        for j in range(chunks_per_w):
            copies.append(
                pltpu.async_copy(
                    table_hbm.at[idx_v.at[j]],
                    rows_v.at[pl.ds(j * 128, 128)],
                    sem,
                )
            )
        for c in copies:
            c.wait()
        pltpu.sync_copy(rows_v, out_hbm.at[pl.ds(wid * rows_per_w, rows_per_w)])

    return gather_kernel(token_table, idx_2d)


def _tc_body(g_ref, pe_ref, seg_ref, segtab_ref, w_ref, b_ref, gamma_ref,
             beta_ref, out_ref):
    x = g_ref[...] + pe_ref[...]
    seg = seg_ref[...]
    for r in range(3):
        mask = jnp.where(seg == r, 1.0, 0.0)
        x = x + mask * segtab_ref[r, :][None, :]
    y = jnp.dot(x, w_ref[...], preferred_element_type=jnp.float32)
    y = y + b_ref[...]
    mu = jnp.mean(y, axis=-1, keepdims=True)
    d = y - mu
    var = jnp.mean(d * d, axis=-1, keepdims=True)
    yn = d * lax.rsqrt(var + _EPS)
    out_ref[...] = yn * gamma_ref[...] + beta_ref[...]


def _tc_fused(g, pe, seg_col, segtab_pad, W, b, gamma, beta, n_rows, s_len):
    blk = 512
    n_blocks = n_rows // blk
    pe_blocks = s_len // blk

    return pl.pallas_call(
        _tc_body,
        grid=(n_blocks,),
        in_specs=[
            pl.BlockSpec((blk, _EMBED), lambda j: (j, 0)),
            pl.BlockSpec((blk, _EMBED), lambda j: (j % pe_blocks, 0)),
            pl.BlockSpec((blk, 1), lambda j: (j, 0)),
            pl.BlockSpec((8, _EMBED), lambda j: (0, 0)),
            pl.BlockSpec((_EMBED, _DMODEL), lambda j: (0, 0)),
            pl.BlockSpec((1, _DMODEL), lambda j: (0, 0)),
            pl.BlockSpec((1, _DMODEL), lambda j: (0, 0)),
            pl.BlockSpec((1, _DMODEL), lambda j: (0, 0)),
        ],
        out_specs=pl.BlockSpec((blk, _DMODEL), lambda j: (j, 0)),
        out_shape=jax.ShapeDtypeStruct((n_rows, _DMODEL), jnp.float32),
    )(g, pe, seg_col, segtab_pad, W, b, gamma, beta)


def kernel(sequence, sequence_segment, token_table, seg_table, W, b, gamma, beta):
    bsz, s_len = sequence.shape
    n_rows = bsz * s_len

    idx_2d = jnp.reshape(sequence.astype(jnp.int32), (n_rows // 128, 128))
    g = _sc_gather(token_table, idx_2d, n_rows)

    pe = jnp.asarray(_sinusoidal_pe_np(_MAXLEN, _EMBED)[:s_len])
    seg_col = jnp.reshape(sequence_segment.astype(jnp.int32), (n_rows, 1))
    segtab_pad = jnp.zeros((8, _EMBED), jnp.float32).at[:3].set(seg_table)

    out = _tc_fused(g, pe, seg_col, segtab_pad, W,
                    jnp.reshape(b, (1, _DMODEL)),
                    jnp.reshape(gamma, (1, _DMODEL)),
                    jnp.reshape(beta, (1, _DMODEL)),
                    n_rows, s_len)
    return jnp.reshape(out, (bsz, s_len, _DMODEL))


# speedup vs baseline: 2.0763x; 1.1035x over previous
import functools

import jax
import jax.numpy as jnp
import numpy as np
from jax import lax
from jax.experimental import pallas as pl
from jax.experimental.pallas import tpu as pltpu
from jax.experimental.pallas import tpu_sc as plsc

_VOCAB = 100000
_EMBED = 128
_DMODEL = 768
_MAXLEN = 2048
_EPS = 1e-5


def _sinusoidal_pe_np(max_len, d):
    pos = np.arange(max_len, dtype=np.float32)[:, None]
    div = np.exp(np.arange(0, d, 2, dtype=np.float32) * (-np.log(10000.0) / d))
    pe = np.zeros((max_len, d), dtype=np.float32)
    pe[:, 0::2] = np.sin(pos * div)
    pe[:, 1::2] = np.cos(pos * div)
    return pe


def _sc_gather(token_table, idx_2d, n_rows):
    info = plsc.get_sparse_core_info()
    nc, ns = info.num_cores, info.num_subcores
    nw = nc * ns
    rows_per_w = n_rows // nw
    chunks_per_w = rows_per_w // 128

    mesh = plsc.VectorSubcoreMesh(core_axis_name="c", subcore_axis_name="s")

    @functools.partial(
        pl.kernel,
        mesh=mesh,
        out_type=jax.ShapeDtypeStruct((n_rows, _EMBED), jnp.float32),
        scratch_types=[
            pltpu.VMEM((chunks_per_w, 128), jnp.int32),
            pltpu.VMEM((rows_per_w, _EMBED), jnp.float32),
            pltpu.SemaphoreType.DMA,
        ],
    )
    def gather_kernel(table_hbm, idx_hbm, out_hbm, idx_v, rows_v, sem):
        wid = lax.axis_index("s") * nc + lax.axis_index("c")
        pltpu.sync_copy(idx_hbm.at[pl.ds(wid * chunks_per_w, chunks_per_w)], idx_v)
        copies = []
        for j in range(chunks_per_w):
            copies.append(
                pltpu.async_copy(
                    table_hbm.at[idx_v.at[j]],
                    rows_v.at[pl.ds(j * 128, 128)],
                    sem,
                )
            )
        for c in copies:
            c.wait()
        pltpu.sync_copy(rows_v, out_hbm.at[pl.ds(wid * rows_per_w, rows_per_w)])

    return gather_kernel(token_table, idx_2d)


_BLK = 1024


def _tc_body(s_len, g_ref, pe_ref, seg_ref, segtab_ref, w_ref, b_ref,
             gamma_ref, beta_ref, out_ref):
    j = pl.program_id(0)
    pe_off = (j % (s_len // _BLK)) * _BLK
    x = g_ref[...] + pe_ref[pl.ds(pe_off, _BLK), :]
    seg = seg_ref[...]
    for r in range(3):
        mask = jnp.where(seg == r, 1.0, 0.0)
        x = x + mask * segtab_ref[r, :][None, :]
    y = jnp.dot(x, w_ref[...], preferred_element_type=jnp.float32)
    y = y + b_ref[...]
    mu = jnp.mean(y, axis=-1, keepdims=True)
    d = y - mu
    var = jnp.mean(d * d, axis=-1, keepdims=True)
    yn = d * lax.rsqrt(var + _EPS)
    out_ref[...] = yn * gamma_ref[...] + beta_ref[...]


def _tc_fused(g, pe, seg_col, segtab_pad, W, b, gamma, beta, n_rows, s_len):
    n_blocks = n_rows // _BLK

    return pl.pallas_call(
        functools.partial(_tc_body, s_len),
        grid=(n_blocks,),
        in_specs=[
            pl.BlockSpec((_BLK, _EMBED), lambda j: (j, 0)),
            pl.BlockSpec((s_len, _EMBED), lambda j: (0, 0)),
            pl.BlockSpec((_BLK, 1), lambda j: (j, 0)),
            pl.BlockSpec((8, _EMBED), lambda j: (0, 0)),
            pl.BlockSpec((_EMBED, _DMODEL), lambda j: (0, 0)),
            pl.BlockSpec((1, _DMODEL), lambda j: (0, 0)),
            pl.BlockSpec((1, _DMODEL), lambda j: (0, 0)),
            pl.BlockSpec((1, _DMODEL), lambda j: (0, 0)),
        ],
        out_specs=pl.BlockSpec((_BLK, _DMODEL), lambda j: (j, 0)),
        out_shape=jax.ShapeDtypeStruct((n_rows, _DMODEL), jnp.float32),
    )(g, pe, seg_col, segtab_pad, W, b, gamma, beta)


def kernel(sequence, sequence_segment, token_table, seg_table, W, b, gamma, beta):
    bsz, s_len = sequence.shape
    n_rows = bsz * s_len

    idx_2d = jnp.reshape(sequence.astype(jnp.int32), (n_rows // 128, 128))
    g = _sc_gather(token_table, idx_2d, n_rows)

    pe = jnp.asarray(_sinusoidal_pe_np(_MAXLEN, _EMBED)[:s_len])
    seg_col = jnp.reshape(sequence_segment.astype(jnp.int32), (n_rows, 1))
    segtab_pad = jnp.zeros((8, _EMBED), jnp.float32).at[:3].set(seg_table)

    out = _tc_fused(g, pe, seg_col, segtab_pad, W,
                    jnp.reshape(b, (1, _DMODEL)),
                    jnp.reshape(gamma, (1, _DMODEL)),
                    jnp.reshape(beta, (1, _DMODEL)),
                    n_rows, s_len)
    return jnp.reshape(out, (bsz, s_len, _DMODEL))
